# baseline (speedup 1.0000x reference)
import functools

import jax
import jax.numpy as jnp
from jax import lax
from jax.experimental import pallas as pl
from jax.experimental.pallas import tpu as pltpu
from jax.experimental.pallas import tpu_sc as plsc

_KB = 16
_LP = 32
_DP = 128


@functools.partial(jax.jit, static_argnums=(2, 3))
def _embed(x, table, nc, ns):
    nw = nc * ns
    b, l = x.shape
    d = table.shape[1]
    bw = b // nw
    n_super = bw // _KB
    n_pairs = n_super // 2
    kb = _KB
    mesh = plsc.VectorSubcoreMesh(core_axis_name="c", subcore_axis_name="s")

    @functools.partial(
        pl.kernel,
        mesh=mesh,
        out_type=jax.ShapeDtypeStruct((b, _LP, _DP), table.dtype),
        compiler_params=pltpu.CompilerParams(use_tc_tiling_on_sc=False),
        scratch_types=[
            pltpu.VMEM((bw, l), jnp.int32),
            pltpu.VMEM((kb * l, d), jnp.float32),
            pltpu.VMEM((kb * l, d), jnp.float32),
            pltpu.SemaphoreType.DMA,
            pltpu.SemaphoreType.DMA,
            pltpu.SemaphoreType.DMA,
            pltpu.SemaphoreType.DMA,
        ],
    )
    def emb(x_hbm, table_hbm, out_hbm, idx_v, buf0, buf1,
            gsem0, gsem1, wsem0, wsem1):
        wid = lax.axis_index("s") * nc + lax.axis_index("c")
        base = wid * bw
        pltpu.sync_copy(x_hbm.at[pl.ds(base, bw)], idx_v)

        def fire_gathers(sg, buf, gsem):
            for j in range(kb):
                pltpu.async_copy(table_hbm.at[idx_v.at[sg * kb + j]],
                                 buf.at[pl.ds(j * l, l)], gsem)

        def drain_gathers(buf, gsem):
            pltpu.make_async_copy(table_hbm.at[pl.ds(0, kb * l)], buf,
                                  gsem).wait()

        def fire_write(sg, buf, wsem):
            for j in range(kb):
                pltpu.async_copy(
                    buf.at[pl.ds(j * l, l)],
                    out_hbm.at[base + sg * kb + j, pl.ds(0, l), pl.ds(0, d)],
                    wsem)

        def drain_write(buf, wsem):
            for j in range(kb):
                pltpu.make_async_copy(
                    buf.at[pl.ds(j * l, l)],
                    out_hbm.at[0, pl.ds(0, l), pl.ds(0, d)], wsem).wait()

        fire_gathers(0, buf0, gsem0)

        def body(t, carry):
            @pl.when(t > 0)
            def _():
                drain_write(buf1, wsem1)

            fire_gathers(2 * t + 1, buf1, gsem1)
            drain_gathers(buf0, gsem0)
            fire_write(2 * t, buf0, wsem0)
            drain_write(buf0, wsem0)

            @pl.when(t < n_pairs - 1)
            def _():
                fire_gathers(2 * t + 2, buf0, gsem0)

            drain_gathers(buf1, gsem1)
            fire_write(2 * t + 1, buf1, wsem1)
            return carry

        lax.fori_loop(0, n_pairs, body, 0)
        drain_write(buf1, wsem1)

    return emb(x, table)


def kernel(x, table):
    info = plsc.get_sparse_core_info()
    nc, ns = info.num_cores, info.num_subcores
    l, d = x.shape[1], table.shape[1]
    padded = _embed(x.astype(jnp.int32), table, nc, ns)
    return padded[:, :l, :d]


# speedup vs baseline: 1.5561x; 1.5561x over previous
import functools

import jax
import jax.numpy as jnp
from jax import lax
from jax.experimental import pallas as pl
from jax.experimental.pallas import tpu as pltpu
from jax.experimental.pallas import tpu_sc as plsc

_KB = 32
_LP = 24
_DP = 128


@functools.partial(jax.jit, static_argnums=(2, 3))
def _embed(x, table, nc, ns):
    nw = nc * ns
    b, l = x.shape
    d = table.shape[1]
    bw = b // nw
    n_super = bw // _KB
    n_pairs = n_super // 2
    kb = _KB
    mesh = plsc.VectorSubcoreMesh(core_axis_name="c", subcore_axis_name="s")

    @functools.partial(
        pl.kernel,
        mesh=mesh,
        out_type=jax.ShapeDtypeStruct((b, _LP, _DP), table.dtype),
        compiler_params=pltpu.CompilerParams(use_tc_tiling_on_sc=False),
        scratch_types=[
            pltpu.VMEM((bw, l), jnp.int32),
            pltpu.VMEM((kb * l, d), jnp.float32),
            pltpu.VMEM((kb * l, d), jnp.float32),
            pltpu.SemaphoreType.DMA,
            pltpu.SemaphoreType.DMA,
            pltpu.SemaphoreType.DMA,
            pltpu.SemaphoreType.DMA,
        ],
    )
    def emb(x_hbm, table_hbm, out_hbm, idx_v, buf0, buf1,
            gsem0, gsem1, wsem0, wsem1):
        wid = lax.axis_index("s") * nc + lax.axis_index("c")
        base = wid * bw
        pltpu.sync_copy(x_hbm.at[pl.ds(base, bw)], idx_v)

        def fire_gathers(sg, buf, gsem):
            for j in range(kb):
                pltpu.async_copy(table_hbm.at[idx_v.at[sg * kb + j]],
                                 buf.at[pl.ds(j * l, l)], gsem)

        def drain_gathers(buf, gsem):
            pltpu.make_async_copy(table_hbm.at[pl.ds(0, kb * l)], buf,
                                  gsem).wait()

        def fire_write(sg, buf, wsem):
            for j in range(kb):
                pltpu.async_copy(
                    buf.at[pl.ds(j * l, l)],
                    out_hbm.at[base + sg * kb + j, pl.ds(0, l), pl.ds(0, d)],
                    wsem)

        def drain_write(buf, wsem):
            for j in range(kb):
                pltpu.make_async_copy(
                    buf.at[pl.ds(j * l, l)],
                    out_hbm.at[0, pl.ds(0, l), pl.ds(0, d)], wsem).wait()

        fire_gathers(0, buf0, gsem0)

        def body(t, carry):
            @pl.when(t > 0)
            def _():
                drain_write(buf1, wsem1)

            fire_gathers(2 * t + 1, buf1, gsem1)
            drain_gathers(buf0, gsem0)
            fire_write(2 * t, buf0, wsem0)
            drain_write(buf0, wsem0)

            @pl.when(t < n_pairs - 1)
            def _():
                fire_gathers(2 * t + 2, buf0, gsem0)

            drain_gathers(buf1, gsem1)
            fire_write(2 * t + 1, buf1, wsem1)
            return carry

        lax.fori_loop(0, n_pairs, body, 0)
        drain_write(buf1, wsem1)

    return emb(x, table)


def kernel(x, table):
    info = plsc.get_sparse_core_info()
    nc, ns = info.num_cores, info.num_subcores
    l, d = x.shape[1], table.shape[1]
    padded = _embed(x.astype(jnp.int32), table, nc, ns)
    return padded[:, :l, :d]


# speedup vs baseline: 1.5593x; 1.0020x over previous
import functools

import jax
import jax.numpy as jnp
from jax import lax
from jax.experimental import pallas as pl
from jax.experimental.pallas import tpu as pltpu
from jax.experimental.pallas import tpu_sc as plsc

_KB = 32
_LP = 24
_DP = 128


@functools.partial(jax.jit, static_argnums=(2, 3))
def _embed(x, table, nc, ns):
    nw = nc * ns
    b, l = x.shape
    d = table.shape[1]
    bw = b // nw
    n_super = bw // _KB
    n_pairs = n_super // 2
    kb = _KB
    mesh = plsc.VectorSubcoreMesh(core_axis_name="c", subcore_axis_name="s")

    @functools.partial(
        pl.kernel,
        mesh=mesh,
        out_type=jax.ShapeDtypeStruct((b, _LP, _DP), table.dtype),
        compiler_params=pltpu.CompilerParams(use_tc_tiling_on_sc=False),
        scratch_types=[
            pltpu.VMEM((bw, l), jnp.int32),
            pltpu.VMEM((kb * l, d), jnp.float32),
            pltpu.VMEM((kb * l, d), jnp.float32),
            pltpu.SemaphoreType.DMA,
            pltpu.SemaphoreType.DMA,
            pltpu.SemaphoreType.DMA,
            pltpu.SemaphoreType.DMA,
        ],
    )
    def emb(x_hbm, table_hbm, out_hbm, idx_v, buf0, buf1,
            gsem0, gsem1, wsem0, wsem1):
        wid = lax.axis_index("s") * nc + lax.axis_index("c")
        base = wid * bw
        pltpu.sync_copy(x_hbm.at[pl.ds(base, bw)], idx_v)

        def fire_gathers(sg, buf, gsem):
            for j in range(kb):
                pltpu.async_copy(table_hbm.at[idx_v.at[sg * kb + j]],
                                 buf.at[pl.ds(j * l, l)], gsem)

        def drain_gathers(buf, gsem):
            pltpu.make_async_copy(table_hbm.at[pl.ds(0, kb * l)], buf,
                                  gsem).wait()

        def fire_write(sg, buf, wsem):
            for j in range(kb):
                pltpu.async_copy(
                    buf.at[pl.ds(j * l, l)],
                    out_hbm.at[base + sg * kb + j, pl.ds(0, l), pl.ds(0, d)],
                    wsem)

        def drain_write(buf, wsem):
            for j in range(kb):
                pltpu.make_async_copy(
                    buf.at[pl.ds(j * l, l)],
                    out_hbm.at[0, pl.ds(0, l), pl.ds(0, d)], wsem).wait()

        fire_gathers(0, buf0, gsem0)

        def body(t, carry):
            @pl.when(t > 0)
            def _():
                drain_write(buf1, wsem1)

            fire_gathers(2 * t + 1, buf1, gsem1)
            drain_gathers(buf0, gsem0)
            fire_write(2 * t, buf0, wsem0)
            drain_write(buf0, wsem0)

            @pl.when(t < n_pairs - 1)
            def _():
                fire_gathers(2 * t + 2, buf0, gsem0)

            drain_gathers(buf1, gsem1)
            fire_write(2 * t + 1, buf1, wsem1)
            return carry

        lax.fori_loop(0, n_pairs, body, 0)
        drain_write(buf1, wsem1)

    return emb(x, table)


def kernel(x, table):
    info = plsc.get_sparse_core_info()
    nc, ns = info.num_cores, info.num_subcores
    l, d = x.shape[1], table.shape[1]
    padded = _embed(x, table, nc, ns)
    return padded[:, :l, :d]
